# baseline (speedup 1.0000x reference)
import functools

import jax
import jax.numpy as jnp
from jax import lax
from jax.experimental import pallas as pl
from jax.experimental.pallas import tpu as pltpu
from jax.experimental.pallas import tpu_sc as plsc

D = 128
V0, V1, V2 = 5, 6, 2
VT = V0 * V1 * V2
VTP = 64

NC, NS = 2, 16
NW = NC * NS

C = 128
NSLOT = 2


def _table_body(w0_ref, w1_ref, w2_ref, t_ref):
    for r in range(VTP):
        q = min(r, VT - 1)
        i0, i1, i2 = q // (V1 * V2), (q // V2) % V1, q % V2
        t_ref[pl.ds(r, 1), :] = (
            w0_ref[pl.ds(i0, 1), :]
            + w1_ref[pl.ds(i1, 1), :]
            + w2_ref[pl.ds(i2, 1), :]
        )


def _build_table(W0, W1, W2):
    return pl.pallas_call(
        _table_body,
        out_shape=jax.ShapeDtypeStruct((VTP, D), jnp.float32),
    )(W0, W1, W2)


def _sc_body(bpw, tab_hbm, ea0_hbm, ea1_hbm, ea2_hbm, out_hbm, t_v, ea0_v,
             ea1_v, ea2_v, rows_v, wsem0, wsem1):
    wid = lax.axis_index("s") * NC + lax.axis_index("c")
    base = wid * bpw

    pltpu.sync_copy(tab_hbm, t_v)
    pltpu.sync_copy(ea0_hbm.at[pl.ds(base, bpw)], ea0_v)
    pltpu.sync_copy(ea1_hbm.at[pl.ds(base, bpw)], ea1_v)
    pltpu.sync_copy(ea2_hbm.at[pl.ds(base, bpw)], ea2_v)

    iota = lax.iota(jnp.int32, 16)
    _dnums = lax.GatherDimensionNumbers(
        offset_dims=(), collapsed_slice_dims=(0,), start_index_map=(0,))

    def lane_bcast(vec, l):
        return lax.gather(
            vec, jnp.full((16, 1), l, jnp.int32), _dnums, slice_sizes=(1,),
            mode=lax.GatherScatterMode.PROMISE_IN_BOUNDS)

    def expand_group(eo, slot, g):
        i0 = ea0_v[pl.ds(eo, 16)]
        i1 = ea1_v[pl.ds(eo, 16)]
        i2 = ea2_v[pl.ds(eo, 16)]
        civ = (i0 * (V1 * V2) + i1 * V2 + i2) * D
        cis = [civ[l] for l in range(16)]
        base_row = slot * C + g * 16
        prev = None
        for l in range(16):
            loads = []
            for c in range(D // 16):
                loads.append(t_v[pl.ds(cis[l] + c * 16, 16)])
                if prev is not None:
                    pv_l, pv = prev
                    rows_v[base_row + pv_l, pl.ds(c * 16, 16)] = pv[c]
            prev = (l, loads)
        pv_l, pv = prev
        for c in range(D // 16):
            rows_v[base_row + pv_l, pl.ds(c * 16, 16)] = pv[c]

    n_blk = bpw // C
    tail = bpw - n_blk * C
    assert n_blk % NSLOT == 0 and tail % 16 == 0

    def wait_write(sem, slot, o):
        pltpu.make_async_copy(
            rows_v.at[pl.ds(slot * C, C)],
            out_hbm.at[pl.ds(base + lax.max(o, 0) * C, C)],
            sem,
        ).wait()

    def pair_body(p, carry):
        for slot, sem in ((0, wsem0), (1, wsem1)):
            o = p * NSLOT + slot

            @pl.when(p > 0)
            def _():
                wait_write(sem, slot, o - NSLOT)

            for g in range(C // 16):
                expand_group(o * C + g * 16, slot, g)
            pltpu.async_copy(
                rows_v.at[pl.ds(slot * C, C)],
                out_hbm.at[pl.ds(base + o * C, C)],
                sem,
            )
        return carry

    lax.fori_loop(0, n_blk // NSLOT, pair_body, 0)
    wait_write(wsem0, 0, n_blk - NSLOT)
    wait_write(wsem1, 1, n_blk - NSLOT + 1)

    if tail:
        for g in range(tail // 16):
            expand_group(n_blk * C + g * 16, 0, g)
        pltpu.sync_copy(
            rows_v.at[pl.ds(0, tail)],
            out_hbm.at[pl.ds(base + n_blk * C, tail)],
        )


def kernel(edge_attr, W0, W1, W2):
    E = edge_attr.shape[0]
    bpw = E // NW
    assert E == bpw * NW and bpw % 16 == 0

    table = _build_table(W0, W1, W2)
    ea = edge_attr.astype(jnp.int32)
    ea0, ea1, ea2 = ea[:, 0], ea[:, 1], ea[:, 2]

    mesh = plsc.VectorSubcoreMesh(core_axis_name="c", subcore_axis_name="s")
    sc_kernel = functools.partial(
        pl.kernel,
        out_type=jax.ShapeDtypeStruct((E, D), jnp.float32),
        mesh=mesh,
        compiler_params=pltpu.CompilerParams(needs_layout_passes=False),
        scratch_types=[
            pltpu.VMEM((VTP * D,), jnp.float32),
            pltpu.VMEM((bpw,), jnp.int32),
            pltpu.VMEM((bpw,), jnp.int32),
            pltpu.VMEM((bpw,), jnp.int32),
            pltpu.VMEM((NSLOT * C, D), jnp.float32),
            pltpu.SemaphoreType.DMA,
            pltpu.SemaphoreType.DMA,
        ],
    )(functools.partial(_sc_body, bpw))
    return sc_kernel(table.reshape(-1), ea0, ea1, ea2)


# speedup vs baseline: 1.6336x; 1.6336x over previous
import functools

import jax
import jax.numpy as jnp
from jax import lax
from jax.experimental import pallas as pl
from jax.experimental.pallas import tpu as pltpu
from jax.experimental.pallas import tpu_sc as plsc

D = 128
V0, V1, V2 = 5, 6, 2
VT = V0 * V1 * V2
VTP = 64

NC, NS = 2, 16
NW = NC * NS

C = 32
NSLOT = 2


def _table_body(w0_ref, w1_ref, w2_ref, t_ref):
    for r in range(VTP):
        q = min(r, VT - 1)
        i0, i1, i2 = q // (V1 * V2), (q // V2) % V1, q % V2
        t_ref[pl.ds(r, 1), :] = (
            w0_ref[pl.ds(i0, 1), :]
            + w1_ref[pl.ds(i1, 1), :]
            + w2_ref[pl.ds(i2, 1), :]
        )


def _build_table(W0, W1, W2):
    return pl.pallas_call(
        _table_body,
        out_shape=jax.ShapeDtypeStruct((VTP, D), jnp.float32),
    )(W0, W1, W2)


def _sc_body(bpw, tab_hbm, ea0_hbm, ea1_hbm, ea2_hbm, out_hbm, t_v, ea0_v,
             ea1_v, ea2_v, rows_v, wsem0, wsem1):
    wid = lax.axis_index("s") * NC + lax.axis_index("c")
    base = wid * bpw

    pltpu.sync_copy(tab_hbm, t_v)
    pltpu.sync_copy(ea0_hbm.at[pl.ds(base, bpw)], ea0_v)
    pltpu.sync_copy(ea1_hbm.at[pl.ds(base, bpw)], ea1_v)
    pltpu.sync_copy(ea2_hbm.at[pl.ds(base, bpw)], ea2_v)

    iota = lax.iota(jnp.int32, 16)
    _dnums = lax.GatherDimensionNumbers(
        offset_dims=(), collapsed_slice_dims=(0,), start_index_map=(0,))

    def lane_bcast(vec, l):
        return lax.gather(
            vec, jnp.full((16, 1), l, jnp.int32), _dnums, slice_sizes=(1,),
            mode=lax.GatherScatterMode.PROMISE_IN_BOUNDS)

    def expand_group(eo, slot, g):
        i0 = ea0_v[pl.ds(eo, 16)]
        i1 = ea1_v[pl.ds(eo, 16)]
        i2 = ea2_v[pl.ds(eo, 16)]
        civ = (i0 * (V1 * V2) + i1 * V2 + i2) * D
        cis = [civ[l] for l in range(16)]
        base_row = slot * C + g * 16
        prev = None
        for l in range(16):
            loads = []
            for c in range(D // 16):
                loads.append(t_v[pl.ds(cis[l] + c * 16, 16)])
                if prev is not None:
                    pv_l, pv = prev
                    rows_v[base_row + pv_l, pl.ds(c * 16, 16)] = pv[c]
            prev = (l, loads)
        pv_l, pv = prev
        for c in range(D // 16):
            rows_v[base_row + pv_l, pl.ds(c * 16, 16)] = pv[c]

    n_blk = bpw // C
    tail = bpw - n_blk * C
    assert n_blk % NSLOT == 0 and tail % 16 == 0

    def wait_write(sem, slot, o):
        pltpu.make_async_copy(
            rows_v.at[pl.ds(slot * C, C)],
            out_hbm.at[pl.ds(base + lax.max(o, 0) * C, C)],
            sem,
        ).wait()

    def pair_body(p, carry):
        for slot, sem in ((0, wsem0), (1, wsem1)):
            o = p * NSLOT + slot

            @pl.when(p > 0)
            def _():
                wait_write(sem, slot, o - NSLOT)

            for g in range(C // 16):
                expand_group(o * C + g * 16, slot, g)
            pltpu.async_copy(
                rows_v.at[pl.ds(slot * C, C)],
                out_hbm.at[pl.ds(base + o * C, C)],
                sem,
            )
        return carry

    lax.fori_loop(0, n_blk // NSLOT, pair_body, 0)
    wait_write(wsem0, 0, n_blk - NSLOT)
    wait_write(wsem1, 1, n_blk - NSLOT + 1)

    if tail:
        for g in range(tail // 16):
            expand_group(n_blk * C + g * 16, 0, g)
        pltpu.sync_copy(
            rows_v.at[pl.ds(0, tail)],
            out_hbm.at[pl.ds(base + n_blk * C, tail)],
        )


def kernel(edge_attr, W0, W1, W2):
    E = edge_attr.shape[0]
    bpw = E // NW
    assert E == bpw * NW and bpw % 16 == 0

    table = _build_table(W0, W1, W2)
    ea = edge_attr.astype(jnp.int32)
    ea0, ea1, ea2 = ea[:, 0], ea[:, 1], ea[:, 2]

    mesh = plsc.VectorSubcoreMesh(core_axis_name="c", subcore_axis_name="s")
    sc_kernel = functools.partial(
        pl.kernel,
        out_type=jax.ShapeDtypeStruct((E, D), jnp.float32),
        mesh=mesh,
        compiler_params=pltpu.CompilerParams(needs_layout_passes=False),
        scratch_types=[
            pltpu.VMEM((VTP * D,), jnp.float32),
            pltpu.VMEM((bpw,), jnp.int32),
            pltpu.VMEM((bpw,), jnp.int32),
            pltpu.VMEM((bpw,), jnp.int32),
            pltpu.VMEM((NSLOT * C, D), jnp.float32),
            pltpu.SemaphoreType.DMA,
            pltpu.SemaphoreType.DMA,
        ],
    )(functools.partial(_sc_body, bpw))
    return sc_kernel(table.reshape(-1), ea0, ea1, ea2)


# speedup vs baseline: 1.7395x; 1.0648x over previous
import functools

import jax
import jax.numpy as jnp
from jax import lax
from jax.experimental import pallas as pl
from jax.experimental.pallas import tpu as pltpu
from jax.experimental.pallas import tpu_sc as plsc

D = 128
V0, V1, V2 = 5, 6, 2
VT = V0 * V1 * V2
VTP = 64

NC, NS = 2, 16
NW = NC * NS

C = 64
NSLOT = 2


def _table_body(w0_ref, w1_ref, w2_ref, t_ref):
    for r in range(VTP):
        q = min(r, VT - 1)
        i0, i1, i2 = q // (V1 * V2), (q // V2) % V1, q % V2
        t_ref[pl.ds(r, 1), :] = (
            w0_ref[pl.ds(i0, 1), :]
            + w1_ref[pl.ds(i1, 1), :]
            + w2_ref[pl.ds(i2, 1), :]
        )


def _build_table(W0, W1, W2):
    return pl.pallas_call(
        _table_body,
        out_shape=jax.ShapeDtypeStruct((VTP, D), jnp.float32),
    )(W0, W1, W2)


def _sc_body(bpw, tab_hbm, ea0_hbm, ea1_hbm, ea2_hbm, out_hbm, t_v, ea0_v,
             ea1_v, ea2_v, rows_v, wsem0, wsem1):
    wid = lax.axis_index("s") * NC + lax.axis_index("c")
    base = wid * bpw

    pltpu.sync_copy(tab_hbm, t_v)
    pltpu.sync_copy(ea0_hbm.at[pl.ds(base, bpw)], ea0_v)
    pltpu.sync_copy(ea1_hbm.at[pl.ds(base, bpw)], ea1_v)
    pltpu.sync_copy(ea2_hbm.at[pl.ds(base, bpw)], ea2_v)

    iota = lax.iota(jnp.int32, 16)
    _dnums = lax.GatherDimensionNumbers(
        offset_dims=(), collapsed_slice_dims=(0,), start_index_map=(0,))

    def lane_bcast(vec, l):
        return lax.gather(
            vec, jnp.full((16, 1), l, jnp.int32), _dnums, slice_sizes=(1,),
            mode=lax.GatherScatterMode.PROMISE_IN_BOUNDS)

    def group_offsets(eo):
        i0 = ea0_v[pl.ds(eo, 16)]
        i1 = ea1_v[pl.ds(eo, 16)]
        i2 = ea2_v[pl.ds(eo, 16)]
        civ = (i0 * (V1 * V2) + i1 * V2 + i2) * D
        return [civ[l] for l in range(16)]

    n_blk = bpw // C
    tail = bpw - n_blk * C
    assert n_blk % NSLOT == 0 and tail % 16 == 0
    CB = NSLOT * C

    def wait_write(sem, slot, o):
        pltpu.make_async_copy(
            rows_v.at[pl.ds(slot * C, C)],
            out_hbm.at[pl.ds(base + lax.max(o, 0) * C, C)],
            sem,
        ).wait()

    def fire_write(sem, slot, o):
        pltpu.async_copy(
            rows_v.at[pl.ds(slot * C, C)],
            out_hbm.at[pl.ds(base + o * C, C)],
            sem,
        )

    def pair_body(p, carry):
        @pl.when(p > 0)
        def _():
            wait_write(wsem0, 0, (p - 1) * NSLOT)

        prev = None
        cis = None
        for e in range(CB):
            if e % 16 == 0:
                cis = group_offsets(p * CB + e)
            if e == C:
                @pl.when(p > 0)
                def _():
                    wait_write(wsem1, 1, (p - 1) * NSLOT + 1)
            loads = []
            for c in range(D // 16):
                loads.append(t_v[pl.ds(cis[e % 16] + c * 16, 16)])
                if prev is not None:
                    pv_e, pv = prev
                    rows_v[pv_e, pl.ds(c * 16, 16)] = pv[c]
            prev = (e, loads)
            if e == C:
                fire_write(wsem0, 0, p * NSLOT)
        pv_e, pv = prev
        for c in range(D // 16):
            rows_v[pv_e, pl.ds(c * 16, 16)] = pv[c]
        fire_write(wsem1, 1, p * NSLOT + 1)
        return carry

    lax.fori_loop(0, n_blk // NSLOT, pair_body, 0)
    wait_write(wsem0, 0, n_blk - NSLOT)
    wait_write(wsem1, 1, n_blk - NSLOT + 1)

    if tail:
        for g in range(tail // 16):
            cis = group_offsets(n_blk * C + g * 16)
            for l in range(16):
                for c in range(D // 16):
                    rows_v[g * 16 + l, pl.ds(c * 16, 16)] = (
                        t_v[pl.ds(cis[l] + c * 16, 16)])
        pltpu.sync_copy(
            rows_v.at[pl.ds(0, tail)],
            out_hbm.at[pl.ds(base + n_blk * C, tail)],
        )


def kernel(edge_attr, W0, W1, W2):
    E = edge_attr.shape[0]
    bpw = E // NW
    assert E == bpw * NW and bpw % 16 == 0

    table = _build_table(W0, W1, W2)
    ea = edge_attr.astype(jnp.int32)
    ea0, ea1, ea2 = ea[:, 0], ea[:, 1], ea[:, 2]

    mesh = plsc.VectorSubcoreMesh(core_axis_name="c", subcore_axis_name="s")
    sc_kernel = functools.partial(
        pl.kernel,
        out_type=jax.ShapeDtypeStruct((E, D), jnp.float32),
        mesh=mesh,
        compiler_params=pltpu.CompilerParams(needs_layout_passes=False),
        scratch_types=[
            pltpu.VMEM((VTP * D,), jnp.float32),
            pltpu.VMEM((bpw,), jnp.int32),
            pltpu.VMEM((bpw,), jnp.int32),
            pltpu.VMEM((bpw,), jnp.int32),
            pltpu.VMEM((NSLOT * C, D), jnp.float32),
            pltpu.SemaphoreType.DMA,
            pltpu.SemaphoreType.DMA,
        ],
    )(functools.partial(_sc_body, bpw))
    return sc_kernel(table.reshape(-1), ea0, ea1, ea2)
